# baseline (speedup 1.0000x reference)
import functools

import jax
import jax.numpy as jnp
import numpy as np
from jax import lax
from jax.experimental import pallas as pl
from jax.experimental.pallas import tpu as pltpu
from jax.experimental.pallas import tpu_sc as plsc

N_NODES = 100000
N_EDGES = 3200000
D = 16
LAYERS = 3

NC = 2
NS = 16
CHUNK = 512
NSETS = 3
CPT0 = 324
CPT1 = 72
EDGES_C0 = CHUNK * CPT0 * NS
E_PAD = CHUNK * (CPT0 + CPT1) * NS
ROWS_PER_TILE = 6256
LAST_ROWS = N_NODES - 15 * ROWS_PER_TILE

_mesh = plsc.VectorSubcoreMesh(core_axis_name="c", subcore_axis_name="s")

_sc_params = pltpu.CompilerParams(
    needs_layout_passes=False, use_tc_tiling_on_sc=False)

_GDN = lax.GatherDimensionNumbers(
    offset_dims=(), collapsed_slice_dims=(0,), start_index_map=(0,))


def _bcast_lane(vec, j):
    idx = (lax.iota(jnp.int32, 16) * 0 + j).reshape(16, 1)
    return lax.gather(vec, idx, _GDN, (1,),
                      mode=lax.GatherScatterMode.PROMISE_IN_BOUNDS)


@functools.partial(
    pl.kernel,
    mesh=_mesh,
    compiler_params=_sc_params,
    out_type=jax.ShapeDtypeStruct((NC, N_NODES, D), jnp.float32),
    scratch_types=(
        [pltpu.VMEM((CHUNK,), jnp.int32)] * NSETS
        + [pltpu.VMEM((CHUNK,), jnp.int32)] * NSETS
        + [pltpu.VMEM((CHUNK,), jnp.float32)] * NSETS
        + [pltpu.VMEM((CHUNK, D), jnp.float32)] * NSETS
        + [pltpu.VMEM_SHARED((N_NODES, D), jnp.float32)]
        + [pltpu.SemaphoreType.DMA] * (3 * NSETS)
    ),
)
def _sc_aggregate(h_hbm, src_hbm, dst_hbm, w_hbm, zeros_hbm, out_hbm, *sc):
    src_v = sc[0:NSETS]
    dst_v = sc[NSETS:2 * NSETS]
    w_v = sc[2 * NSETS:3 * NSETS]
    rows_v = sc[3 * NSETS:4 * NSETS]
    acc_sh = sc[4 * NSETS]
    sem_i = sc[4 * NSETS + 1:4 * NSETS + 1 + NSETS]
    sem_g = sc[4 * NSETS + 1 + NSETS:4 * NSETS + 1 + 2 * NSETS]
    sem_s = sc[4 * NSETS + 1 + 2 * NSETS:4 * NSETS + 1 + 3 * NSETS]

    core = lax.axis_index("c")
    sid = lax.axis_index("s")
    wid = core * NS + sid

    row0 = pl.multiple_of(sid * ROWS_PER_TILE, 8)

    @pl.when(sid < NS - 1)
    def _():
        pltpu.sync_copy(zeros_hbm.at[pl.ds(row0, ROWS_PER_TILE)],
                        acc_sh.at[pl.ds(row0, ROWS_PER_TILE)])

    @pl.when(sid == NS - 1)
    def _():
        pltpu.sync_copy(zeros_hbm.at[pl.ds(row0, LAST_ROWS)],
                        acc_sh.at[pl.ds(row0, LAST_ROWS)])

    plsc.subcore_barrier()

    my_cpt = jnp.where(core == 0, CPT0, CPT1)
    ebase = jnp.where(core == 0,
                      sid * (CPT0 * CHUNK),
                      EDGES_C0 + sid * (CPT1 * CHUNK))

    def fire_idx(s, ci):
        eb = pl.multiple_of(ebase + ci * CHUNK, CHUNK)
        pltpu.async_copy(src_hbm.at[pl.ds(eb, CHUNK)], src_v[s], sem_i[s])
        pltpu.async_copy(dst_hbm.at[pl.ds(eb, CHUNK)], dst_v[s], sem_i[s])
        pltpu.async_copy(w_hbm.at[pl.ds(eb, CHUNK)], w_v[s], sem_i[s])

    def wait_idx(s):
        pltpu.make_async_copy(src_hbm.at[pl.ds(0, CHUNK)], src_v[s],
                              sem_i[s]).wait()
        pltpu.make_async_copy(dst_hbm.at[pl.ds(0, CHUNK)], dst_v[s],
                              sem_i[s]).wait()
        pltpu.make_async_copy(w_hbm.at[pl.ds(0, CHUNK)], w_v[s],
                              sem_i[s]).wait()

    def fire_gather(s):
        pltpu.async_copy(h_hbm.at[src_v[s]], rows_v[s], sem_g[s])

    def wait_gather(s):
        pltpu.make_async_copy(h_hbm.at[src_v[s]], rows_v[s], sem_g[s]).wait()

    def fire_scatter(s):
        pltpu.async_copy(rows_v[s], acc_sh.at[dst_v[s]], sem_s[s], add=True)

    def wait_scatter(s):
        pltpu.make_async_copy(rows_v[s], acc_sh.at[dst_v[s]], sem_s[s]).wait()

    def compute(s):
        @plsc.parallel_loop(0, CHUNK // 16, unroll=2)
        def _grp(g):
            goff = g * 16
            wg = w_v[s][pl.ds(goff, 16)]
            for j in range(16):
                wj = _bcast_lane(wg, j)
                e = goff + j
                rows_v[s][e] = rows_v[s][e] * wj

    fire_idx(0, 0)
    fire_idx(1, 1)
    wait_idx(0)
    fire_gather(0)

    @pl.loop(0, my_cpt // NSETS)
    def _trip(p):
        for j in range(NSETS):
            ci = p * NSETS + j
            b = j
            bn = (j + 1) % NSETS
            bp = (j + 2) % NSETS

            @pl.when(ci + 1 < my_cpt)
            def _():
                wait_idx(bn)
                fire_gather(bn)

            @pl.when(ci >= 1)
            def _():
                wait_scatter(bp)

            @pl.when(ci + 2 < my_cpt)
            def _():
                fire_idx(bp, ci + 2)

            wait_gather(b)
            compute(b)
            fire_scatter(b)

    wait_scatter(2)
    plsc.subcore_barrier()

    @pl.when(sid < NS - 1)
    def _():
        pltpu.sync_copy(acc_sh.at[pl.ds(row0, ROWS_PER_TILE)],
                        out_hbm.at[core, pl.ds(row0, ROWS_PER_TILE)])

    @pl.when(sid == NS - 1)
    def _():
        pltpu.sync_copy(acc_sh.at[pl.ds(row0, LAST_ROWS)],
                        out_hbm.at[core, pl.ds(row0, LAST_ROWS)])


_NF = N_NODES * D // 128
_RB = _NF


def _combine_body(p0_ref, p1_ref, h_ref, wrel_ref, wroot_ref, b_ref, out_ref):
    agg = p0_ref[...] + p1_ref[...]
    out_ref[...] = (
        jnp.dot(agg, wrel_ref[...], preferred_element_type=jnp.float32)
        + jnp.dot(h_ref[...], wroot_ref[...], preferred_element_type=jnp.float32)
        + b_ref[...]
    )


def _tc_combine(p0, p1, h, wrel128, wroot128, b128):
    return pl.pallas_call(
        _combine_body,
        out_shape=jax.ShapeDtypeStruct((_NF, 128), jnp.float32),
        grid=(_NF // _RB,),
        in_specs=[
            pl.BlockSpec((_RB, 128), lambda i: (i, 0)),
            pl.BlockSpec((_RB, 128), lambda i: (i, 0)),
            pl.BlockSpec((_RB, 128), lambda i: (i, 0)),
            pl.BlockSpec((128, 128), lambda i: (0, 0)),
            pl.BlockSpec((128, 128), lambda i: (0, 0)),
            pl.BlockSpec((1, 128), lambda i: (0, 0)),
        ],
        out_specs=pl.BlockSpec((_RB, 128), lambda i: (i, 0)),
    )(p0, p1, h, wrel128, wroot128, b128)


def kernel(x, edge_index, edge_weight, W_rel, b_rel, W_root):
    src = edge_index[0].astype(jnp.int32)
    dst = edge_index[1].astype(jnp.int32)
    w = edge_weight.astype(jnp.float32)

    pad = E_PAD - N_EDGES
    src2 = jnp.concatenate([src, jnp.zeros((pad,), jnp.int32)])
    dst2 = jnp.concatenate([dst, jnp.zeros((pad,), jnp.int32)])
    w_p = jnp.concatenate([w, jnp.zeros((pad,), jnp.float32)])
    zeros = jnp.zeros((N_NODES, D), jnp.float32)

    eye8 = jnp.eye(8, dtype=jnp.float32)
    h = x
    for i in range(LAYERS):
        parts = _sc_aggregate(h, src2, dst2, w_p, zeros)
        wrel128 = jnp.kron(eye8, W_rel[i].T)
        wroot128 = jnp.kron(eye8, W_root[i].T)
        b128 = jnp.tile(b_rel[i], 8).reshape(1, 128)
        h_flat = _tc_combine(parts[0].reshape(_NF, 128),
                             parts[1].reshape(_NF, 128),
                             h.reshape(_NF, 128),
                             wrel128, wroot128, b128)
        h = h_flat.reshape(N_NODES, D)
    return h


# speedup vs baseline: 1.0858x; 1.0858x over previous
import functools

import jax
import jax.numpy as jnp
import numpy as np
from jax import lax
from jax.experimental import pallas as pl
from jax.experimental.pallas import tpu as pltpu
from jax.experimental.pallas import tpu_sc as plsc

N_NODES = 100000
N_EDGES = 3200000
D = 16
LAYERS = 3

NC = 2
NS = 16
CHUNK = 512
NSETS = 3
CPT0 = 330
CPT1 = 66
EDGES_C0 = CHUNK * CPT0 * NS
E_PAD = CHUNK * (CPT0 + CPT1) * NS
ROWS_PER_TILE = 6256
LAST_ROWS = N_NODES - 15 * ROWS_PER_TILE

_mesh = plsc.VectorSubcoreMesh(core_axis_name="c", subcore_axis_name="s")

_sc_params = pltpu.CompilerParams(
    needs_layout_passes=False, use_tc_tiling_on_sc=False)

_GDN = lax.GatherDimensionNumbers(
    offset_dims=(), collapsed_slice_dims=(0,), start_index_map=(0,))


def _bcast_lane(vec, j):
    idx = (lax.iota(jnp.int32, 16) * 0 + j).reshape(16, 1)
    return lax.gather(vec, idx, _GDN, (1,),
                      mode=lax.GatherScatterMode.PROMISE_IN_BOUNDS)


@functools.partial(
    pl.kernel,
    mesh=_mesh,
    compiler_params=_sc_params,
    out_type=jax.ShapeDtypeStruct((NC, N_NODES, D), jnp.float32),
    scratch_types=(
        [pltpu.VMEM((CHUNK,), jnp.int32)] * NSETS
        + [pltpu.VMEM((CHUNK,), jnp.int32)] * NSETS
        + [pltpu.VMEM((CHUNK,), jnp.float32)] * NSETS
        + [pltpu.VMEM((CHUNK, D), jnp.float32)] * NSETS
        + [pltpu.VMEM_SHARED((N_NODES, D), jnp.float32)]
        + [pltpu.SemaphoreType.DMA] * (3 * NSETS)
    ),
)
def _sc_aggregate(h_hbm, src_hbm, dst_hbm, w_hbm, zeros_hbm, out_hbm, *sc):
    src_v = sc[0:NSETS]
    dst_v = sc[NSETS:2 * NSETS]
    w_v = sc[2 * NSETS:3 * NSETS]
    rows_v = sc[3 * NSETS:4 * NSETS]
    acc_sh = sc[4 * NSETS]
    sem_i = sc[4 * NSETS + 1:4 * NSETS + 1 + NSETS]
    sem_g = sc[4 * NSETS + 1 + NSETS:4 * NSETS + 1 + 2 * NSETS]
    sem_s = sc[4 * NSETS + 1 + 2 * NSETS:4 * NSETS + 1 + 3 * NSETS]

    core = lax.axis_index("c")
    sid = lax.axis_index("s")
    wid = core * NS + sid

    row0 = pl.multiple_of(sid * ROWS_PER_TILE, 8)

    @pl.when(sid < NS - 1)
    def _():
        pltpu.sync_copy(zeros_hbm.at[pl.ds(row0, ROWS_PER_TILE)],
                        acc_sh.at[pl.ds(row0, ROWS_PER_TILE)])

    @pl.when(sid == NS - 1)
    def _():
        pltpu.sync_copy(zeros_hbm.at[pl.ds(row0, LAST_ROWS)],
                        acc_sh.at[pl.ds(row0, LAST_ROWS)])

    plsc.subcore_barrier()

    my_cpt = jnp.where(core == 0, CPT0, CPT1)
    ebase = jnp.where(core == 0,
                      sid * (CPT0 * CHUNK),
                      EDGES_C0 + sid * (CPT1 * CHUNK))

    def fire_idx(s, ci):
        eb = pl.multiple_of(ebase + ci * CHUNK, CHUNK)
        pltpu.async_copy(src_hbm.at[pl.ds(eb, CHUNK)], src_v[s], sem_i[s])
        pltpu.async_copy(dst_hbm.at[pl.ds(eb, CHUNK)], dst_v[s], sem_i[s])
        pltpu.async_copy(w_hbm.at[pl.ds(eb, CHUNK)], w_v[s], sem_i[s])

    def wait_idx(s):
        pltpu.make_async_copy(src_hbm.at[pl.ds(0, CHUNK)], src_v[s],
                              sem_i[s]).wait()
        pltpu.make_async_copy(dst_hbm.at[pl.ds(0, CHUNK)], dst_v[s],
                              sem_i[s]).wait()
        pltpu.make_async_copy(w_hbm.at[pl.ds(0, CHUNK)], w_v[s],
                              sem_i[s]).wait()

    def fire_gather(s):
        pltpu.async_copy(h_hbm.at[src_v[s]], rows_v[s], sem_g[s])

    def wait_gather(s):
        pltpu.make_async_copy(h_hbm.at[src_v[s]], rows_v[s], sem_g[s]).wait()

    def fire_scatter(s):
        pltpu.async_copy(rows_v[s], acc_sh.at[dst_v[s]], sem_s[s], add=True)

    def wait_scatter(s):
        pltpu.make_async_copy(rows_v[s], acc_sh.at[dst_v[s]], sem_s[s]).wait()

    def compute(s):
        @plsc.parallel_loop(0, CHUNK // 16, unroll=2)
        def _grp(g):
            goff = g * 16
            wg = w_v[s][pl.ds(goff, 16)]
            for j in range(16):
                wj = _bcast_lane(wg, j)
                e = goff + j
                rows_v[s][e] = rows_v[s][e] * wj

    fire_idx(0, 0)
    fire_idx(1, 1)
    wait_idx(0)
    fire_gather(0)

    @pl.loop(0, my_cpt // NSETS)
    def _trip(p):
        for j in range(NSETS):
            ci = p * NSETS + j
            b = j
            bn = (j + 1) % NSETS
            bp = (j + 2) % NSETS

            @pl.when(ci + 1 < my_cpt)
            def _():
                wait_idx(bn)
                fire_gather(bn)

            @pl.when(ci >= 1)
            def _():
                wait_scatter(bp)

            @pl.when(ci + 2 < my_cpt)
            def _():
                fire_idx(bp, ci + 2)

            wait_gather(b)
            compute(b)
            fire_scatter(b)

    wait_scatter(2)
    plsc.subcore_barrier()

    @pl.when(sid < NS - 1)
    def _():
        pltpu.sync_copy(acc_sh.at[pl.ds(row0, ROWS_PER_TILE)],
                        out_hbm.at[core, pl.ds(row0, ROWS_PER_TILE)])

    @pl.when(sid == NS - 1)
    def _():
        pltpu.sync_copy(acc_sh.at[pl.ds(row0, LAST_ROWS)],
                        out_hbm.at[core, pl.ds(row0, LAST_ROWS)])


def _combine_body(parts_ref, h_ref, wrel_t_ref, wroot_t_ref, b_ref, out_ref):
    agg = parts_ref[0] + parts_ref[1]
    out_ref[...] = (
        jnp.dot(agg, wrel_t_ref[...], preferred_element_type=jnp.float32)
        + jnp.dot(h_ref[...], wroot_t_ref[...], preferred_element_type=jnp.float32)
        + b_ref[...]
    )


_RB = 10000


def _tc_combine(parts, h, wrel_t, wroot_t, b):
    return pl.pallas_call(
        _combine_body,
        out_shape=jax.ShapeDtypeStruct((N_NODES, D), jnp.float32),
        grid=(N_NODES // _RB,),
        in_specs=[
            pl.BlockSpec((NC, _RB, D), lambda i: (0, i, 0)),
            pl.BlockSpec((_RB, D), lambda i: (i, 0)),
            pl.BlockSpec((D, D), lambda i: (0, 0)),
            pl.BlockSpec((D, D), lambda i: (0, 0)),
            pl.BlockSpec((1, D), lambda i: (0, 0)),
        ],
        out_specs=pl.BlockSpec((_RB, D), lambda i: (i, 0)),
    )(parts, h, wrel_t, wroot_t, b)


def kernel(x, edge_index, edge_weight, W_rel, b_rel, W_root):
    src = edge_index[0].astype(jnp.int32)
    dst = edge_index[1].astype(jnp.int32)
    w = edge_weight.astype(jnp.float32)

    pad = E_PAD - N_EDGES
    src2 = jnp.concatenate([src, jnp.zeros((pad,), jnp.int32)])
    dst2 = jnp.concatenate([dst, jnp.zeros((pad,), jnp.int32)])
    w_p = jnp.concatenate([w, jnp.zeros((pad,), jnp.float32)])
    zeros = jnp.zeros((N_NODES, D), jnp.float32)

    h = x
    for i in range(LAYERS):
        parts = _sc_aggregate(h, src2, dst2, w_p, zeros)
        h = _tc_combine(parts, h, W_rel[i].T, W_root[i].T,
                        b_rel[i].reshape(1, D))
    return h


# speedup vs baseline: 1.0950x; 1.0085x over previous
import functools

import jax
import jax.numpy as jnp
import numpy as np
from jax import lax
from jax.experimental import pallas as pl
from jax.experimental.pallas import tpu as pltpu
from jax.experimental.pallas import tpu_sc as plsc

N_NODES = 100000
N_EDGES = 3200000
D = 16
LAYERS = 3

NC = 2
NS = 16
CHUNK = 512
NSETS = 3
CPT0 = 330
CPT1 = 66
EDGES_C0 = CHUNK * CPT0 * NS
E_PAD = CHUNK * (CPT0 + CPT1) * NS
ROWS_PER_TILE = 6256
LAST_ROWS = N_NODES - 15 * ROWS_PER_TILE

_mesh = plsc.VectorSubcoreMesh(core_axis_name="c", subcore_axis_name="s")

_sc_params = pltpu.CompilerParams(
    needs_layout_passes=False, use_tc_tiling_on_sc=False)

_GDN = lax.GatherDimensionNumbers(
    offset_dims=(), collapsed_slice_dims=(0,), start_index_map=(0,))


def _bcast_lane(vec, j):
    idx = (lax.iota(jnp.int32, 16) * 0 + j).reshape(16, 1)
    return lax.gather(vec, idx, _GDN, (1,),
                      mode=lax.GatherScatterMode.PROMISE_IN_BOUNDS)


@functools.partial(
    pl.kernel,
    mesh=_mesh,
    compiler_params=_sc_params,
    out_type=jax.ShapeDtypeStruct((NC, N_NODES, D), jnp.float32),
    scratch_types=(
        [pltpu.VMEM((CHUNK,), jnp.int32)] * NSETS
        + [pltpu.VMEM((CHUNK,), jnp.int32)] * NSETS
        + [pltpu.VMEM((CHUNK,), jnp.float32)] * NSETS
        + [pltpu.VMEM((CHUNK, D), jnp.float32)] * NSETS
        + [pltpu.VMEM_SHARED((N_NODES, D), jnp.float32)]
        + [pltpu.SemaphoreType.DMA] * (3 * NSETS)
    ),
)
def _sc_aggregate(h_hbm, src_hbm, dst_hbm, w_hbm, zeros_hbm, out_hbm, *sc):
    src_v = sc[0:NSETS]
    dst_v = sc[NSETS:2 * NSETS]
    w_v = sc[2 * NSETS:3 * NSETS]
    rows_v = sc[3 * NSETS:4 * NSETS]
    acc_sh = sc[4 * NSETS]
    sem_i = sc[4 * NSETS + 1:4 * NSETS + 1 + NSETS]
    sem_g = sc[4 * NSETS + 1 + NSETS:4 * NSETS + 1 + 2 * NSETS]
    sem_s = sc[4 * NSETS + 1 + 2 * NSETS:4 * NSETS + 1 + 3 * NSETS]

    core = lax.axis_index("c")
    sid = lax.axis_index("s")
    wid = core * NS + sid

    row0 = pl.multiple_of(sid * ROWS_PER_TILE, 8)

    @pl.when(sid < NS - 1)
    def _():
        pltpu.sync_copy(zeros_hbm.at[pl.ds(row0, ROWS_PER_TILE)],
                        acc_sh.at[pl.ds(row0, ROWS_PER_TILE)])

    @pl.when(sid == NS - 1)
    def _():
        pltpu.sync_copy(zeros_hbm.at[pl.ds(row0, LAST_ROWS)],
                        acc_sh.at[pl.ds(row0, LAST_ROWS)])

    plsc.subcore_barrier()

    my_cpt = jnp.where(core == 0, CPT0, CPT1)
    ebase = jnp.where(core == 0,
                      sid * (CPT0 * CHUNK),
                      EDGES_C0 + sid * (CPT1 * CHUNK))

    def fire_idx(s, ci):
        eb = pl.multiple_of(ebase + ci * CHUNK, CHUNK)
        pltpu.async_copy(src_hbm.at[pl.ds(eb, CHUNK)], src_v[s], sem_i[s])
        pltpu.async_copy(dst_hbm.at[pl.ds(eb, CHUNK)], dst_v[s], sem_i[s])
        pltpu.async_copy(w_hbm.at[pl.ds(eb, CHUNK)], w_v[s], sem_i[s])

    def wait_idx(s):
        pltpu.make_async_copy(src_hbm.at[pl.ds(0, CHUNK)], src_v[s],
                              sem_i[s]).wait()
        pltpu.make_async_copy(dst_hbm.at[pl.ds(0, CHUNK)], dst_v[s],
                              sem_i[s]).wait()
        pltpu.make_async_copy(w_hbm.at[pl.ds(0, CHUNK)], w_v[s],
                              sem_i[s]).wait()

    def fire_gather(s):
        @pl.when(core == 0)
        def _():
            pltpu.async_copy(h_hbm.at[src_v[s]], rows_v[s], sem_g[s])

        @pl.when(core == 1)
        def _():
            pltpu.async_copy(h_hbm.at[src_v[s]], rows_v[s], sem_g[s],
                             priority=1)

    def wait_gather(s):
        pltpu.make_async_copy(h_hbm.at[src_v[s]], rows_v[s], sem_g[s]).wait()

    def fire_scatter(s):
        pltpu.async_copy(rows_v[s], acc_sh.at[dst_v[s]], sem_s[s], add=True)

    def wait_scatter(s):
        pltpu.make_async_copy(rows_v[s], acc_sh.at[dst_v[s]], sem_s[s]).wait()

    def compute(s):
        @plsc.parallel_loop(0, CHUNK // 16, unroll=2)
        def _grp(g):
            goff = g * 16
            wg = w_v[s][pl.ds(goff, 16)]
            for j in range(16):
                wj = _bcast_lane(wg, j)
                e = goff + j
                rows_v[s][e] = rows_v[s][e] * wj

    fire_idx(0, 0)
    fire_idx(1, 1)
    wait_idx(0)
    fire_gather(0)

    @pl.loop(0, my_cpt // NSETS)
    def _trip(p):
        for j in range(NSETS):
            ci = p * NSETS + j
            b = j
            bn = (j + 1) % NSETS
            bp = (j + 2) % NSETS

            @pl.when(ci + 1 < my_cpt)
            def _():
                wait_idx(bn)
                fire_gather(bn)

            @pl.when(ci >= 1)
            def _():
                wait_scatter(bp)

            @pl.when(ci + 2 < my_cpt)
            def _():
                fire_idx(bp, ci + 2)

            wait_gather(b)
            compute(b)
            fire_scatter(b)

    wait_scatter(2)
    plsc.subcore_barrier()

    @pl.when(sid < NS - 1)
    def _():
        pltpu.sync_copy(acc_sh.at[pl.ds(row0, ROWS_PER_TILE)],
                        out_hbm.at[core, pl.ds(row0, ROWS_PER_TILE)])

    @pl.when(sid == NS - 1)
    def _():
        pltpu.sync_copy(acc_sh.at[pl.ds(row0, LAST_ROWS)],
                        out_hbm.at[core, pl.ds(row0, LAST_ROWS)])


def _combine_body(parts_ref, h_ref, wrel_t_ref, wroot_t_ref, b_ref, out_ref):
    agg = parts_ref[0] + parts_ref[1]
    out_ref[...] = (
        jnp.dot(agg, wrel_t_ref[...], preferred_element_type=jnp.float32)
        + jnp.dot(h_ref[...], wroot_t_ref[...], preferred_element_type=jnp.float32)
        + b_ref[...]
    )


_RB = 10000


def _tc_combine(parts, h, wrel_t, wroot_t, b):
    return pl.pallas_call(
        _combine_body,
        out_shape=jax.ShapeDtypeStruct((N_NODES, D), jnp.float32),
        grid=(N_NODES // _RB,),
        in_specs=[
            pl.BlockSpec((NC, _RB, D), lambda i: (0, i, 0)),
            pl.BlockSpec((_RB, D), lambda i: (i, 0)),
            pl.BlockSpec((D, D), lambda i: (0, 0)),
            pl.BlockSpec((D, D), lambda i: (0, 0)),
            pl.BlockSpec((1, D), lambda i: (0, 0)),
        ],
        out_specs=pl.BlockSpec((_RB, D), lambda i: (i, 0)),
    )(parts, h, wrel_t, wroot_t, b)


def kernel(x, edge_index, edge_weight, W_rel, b_rel, W_root):
    src = edge_index[0].astype(jnp.int32)
    dst = edge_index[1].astype(jnp.int32)
    w = edge_weight.astype(jnp.float32)

    pad = E_PAD - N_EDGES
    src2 = jnp.concatenate([src, jnp.zeros((pad,), jnp.int32)])
    dst2 = jnp.concatenate([dst, jnp.zeros((pad,), jnp.int32)])
    w_p = jnp.concatenate([w, jnp.zeros((pad,), jnp.float32)])
    zeros = jnp.zeros((N_NODES, D), jnp.float32)

    h = x
    for i in range(LAYERS):
        parts = _sc_aggregate(h, src2, dst2, w_p, zeros)
        h = _tc_combine(parts, h, W_rel[i].T, W_root[i].T,
                        b_rel[i].reshape(1, D))
    return h
